# baseline (speedup 1.0000x reference)
import functools

import jax
import jax.numpy as jnp
from jax import lax
from jax.experimental import pallas as pl
from jax.experimental.pallas import tpu as pltpu
from jax.experimental.pallas import tpu_sc as plsc

VOCAB = 1000000
SEQ_LEN = 200
EMBED_DIM = 64
BATCH = 1024

NUM_WORKERS = 32
TOTAL = BATCH * SEQ_LEN
PER_WORKER = TOTAL // NUM_WORKERS
CHUNK = 800
NCHUNK = PER_WORKER // CHUNK
REPS = CHUNK // SEQ_LEN
COLS = EMBED_DIM // 16

GATHER_SPLITS = [(off, min(128, CHUNK - off)) for off in range(0, CHUNK, 128)]


def _body(x_hbm, tok_hbm, pos_hbm, out_hbm, idx_v, rows_v, pos_v, sem):
    nc = 2
    wid = lax.axis_index("s") * nc + lax.axis_index("c")
    base = wid * PER_WORKER

    pltpu.sync_copy(pos_hbm, pos_v)

    def chunk_body(ci, carry):
        start = base + ci * CHUNK
        pltpu.sync_copy(x_hbm.at[pl.ds(start, CHUNK)], idx_v)

        descs = []
        for off, n in GATHER_SPLITS:
            descs.append(
                pltpu.async_copy(
                    tok_hbm.at[idx_v.at[pl.ds(off, n)]],
                    rows_v.at[pl.ds(off, n)],
                    sem,
                )
            )
        for d in descs:
            d.wait()

        def add_body(r, carry2):
            for c in range(COLS):
                sl = pl.ds(c * 16, 16)
                p = pos_v[r, sl]
                for rep in range(REPS):
                    plsc.addupdate(rows_v.at[rep * SEQ_LEN + r, sl], p)
            return carry2

        lax.fori_loop(0, SEQ_LEN, add_body, 0, unroll=2)

        pltpu.sync_copy(rows_v, out_hbm.at[pl.ds(start, CHUNK)])
        return carry

    lax.fori_loop(0, NCHUNK, chunk_body, 0)


@jax.jit
def _run(x_flat, token_table, pos_table):
    mesh = plsc.VectorSubcoreMesh(core_axis_name="c", subcore_axis_name="s")
    return pl.kernel(
        _body,
        out_type=jax.ShapeDtypeStruct((TOTAL, EMBED_DIM), jnp.float32),
        mesh=mesh,
        scratch_types=[
            pltpu.VMEM((CHUNK,), jnp.int32),
            pltpu.VMEM((CHUNK, EMBED_DIM), jnp.float32),
            pltpu.VMEM((SEQ_LEN, EMBED_DIM), jnp.float32),
            pltpu.SemaphoreType.DMA,
        ],
        compiler_params=pltpu.CompilerParams(use_tc_tiling_on_sc=False),
    )(x_flat, token_table, pos_table)


def kernel(x, token_table, pos_table):
    x_flat = x.reshape(TOTAL).astype(jnp.int32)
    out = _run(x_flat, token_table, pos_table)
    return out.reshape(BATCH, SEQ_LEN, EMBED_DIM)
